# Initial kernel scaffold; baseline (speedup 1.0000x reference)
import jax
import jax.numpy as jnp
from jax.experimental import pallas as pl


def kernel(x, edge_index, batch, params):
    raise NotImplementedError("write your pallas kernel here")


# speedup vs baseline: 6.8096x; 6.8096x over previous
import functools

import jax
import jax.numpy as jnp
from jax import lax
from jax.experimental import pallas as pl
from jax.experimental.pallas import tpu as pltpu
from jax.experimental.pallas import tpu_sc as plsc

_NC = 2
_NS = 16
_W = 80


def _make_seg_sum(n, d, e):
    nworker = _NC * _NS
    ept = e // nworker
    wpt = ept // _W
    zc = 80
    nchunks = n // zc
    nz = -(-nchunks // _NS)
    assert ept % _W == 0 and n % zc == 0 and wpt % 2 == 1

    mesh = plsc.VectorSubcoreMesh(
        core_axis_name="c", subcore_axis_name="s",
        num_cores=_NC, num_subcores=_NS)

    @functools.partial(
        pl.kernel,
        out_type=jax.ShapeDtypeStruct((_NC * n, d), jnp.float32),
        mesh=mesh,
        scratch_types=[
            pltpu.VMEM((ept,), jnp.int32),
            pltpu.VMEM((_W,), jnp.int32),
            pltpu.VMEM((_W,), jnp.int32),
            pltpu.VMEM((_W, d), jnp.float32),
            pltpu.VMEM((_W, d), jnp.float32),
            pltpu.VMEM((zc, d), jnp.float32),
            pltpu.VMEM_SHARED((n, d), jnp.float32),
            pltpu.SemaphoreType.DMA,
            pltpu.SemaphoreType.DMA,
            pltpu.SemaphoreType.DMA,
            pltpu.SemaphoreType.DMA,
        ],
    )
    def seg_sum(x_hbm, src_hbm, dst_hbm, out_hbm,
                sidx, dwin0, dwin1, rows0, rows1, zbuf, acc,
                sem0, sem1, dsem0, dsem1):
        c = lax.axis_index("c")
        s = lax.axis_index("s")
        wid = c * _NS + s
        ebase = wid * ept

        pltpu.sync_copy(src_hbm.at[pl.ds(ebase, ept)], sidx)

        z16 = jnp.zeros((16,), jnp.float32)

        def zrow(i, carry):
            for j in range(d // 16):
                zbuf[i, pl.ds(j * 16, 16)] = z16
            return carry

        lax.fori_loop(0, zc, zrow, 0)
        for k in range(nz):
            cid = s + k * _NS

            @pl.when(cid < nchunks)
            def _():
                pltpu.sync_copy(zbuf, acc.at[pl.ds(cid * zc, zc)])
        plsc.subcore_barrier()

        def gather(w, rows, dwin, sem, dsem):
            pltpu.async_copy(
                x_hbm.at[sidx.at[pl.ds(w * _W, _W)]], rows, sem)
            pltpu.async_copy(
                dst_hbm.at[pl.ds(ebase + w * _W, _W)], dwin, dsem)

        def wait_gather(w, rows, dwin, sem, dsem):
            pltpu.make_async_copy(
                x_hbm.at[sidx.at[pl.ds(w * _W, _W)]], rows, sem).wait()
            pltpu.make_async_copy(
                dst_hbm.at[pl.ds(ebase + w * _W, _W)], dwin, dsem).wait()

        def scatter_add(rows, dwin):
            pltpu.sync_copy(rows, acc.at[dwin], add=True)

        def body(w, carry):
            gather(w, rows0, dwin0, sem0, dsem0)
            wait_gather(w, rows0, dwin0, sem0, dsem0)
            scatter_add(rows0, dwin0)
            return carry

        lax.fori_loop(0, wpt, body, 0)

        plsc.subcore_barrier()
        for k in range(nz):
            cid = s + k * _NS

            @pl.when(cid < nchunks)
            def _():
                pltpu.sync_copy(acc.at[pl.ds(cid * zc, zc)],
                                out_hbm.at[pl.ds(c * n + cid * zc, zc)])

    return seg_sum


def _bn_relu(a, g, b):
    m = jnp.mean(a, axis=0, keepdims=True)
    v = jnp.mean((a - m) ** 2, axis=0, keepdims=True)
    return jnp.maximum(g * (a - m) * lax.rsqrt(v + 1e-5) + b, 0.0)


def _mlp_body(h_ref, agg_ref, w1_ref, v1_ref, w2_ref, v2_ref, out_ref):
    n = h_ref.shape[0]
    agg = agg_ref[...]
    u = h_ref[...] + agg[:n] + agg[n:]
    a = jnp.dot(u, w1_ref[...], preferred_element_type=jnp.float32)
    a = _bn_relu(a + v1_ref[0:1], v1_ref[1:2], v1_ref[2:3])
    a = jnp.dot(a, w2_ref[...], preferred_element_type=jnp.float32)
    out_ref[...] = _bn_relu(a + v2_ref[0:1], v2_ref[1:2], v2_ref[2:3])


def _mlp_pool_body(h_ref, agg_ref, w1_ref, v1_ref, w2_ref, v2_ref,
                   batch_ref, wc_ref, bc_ref, out_ref):
    n = h_ref.shape[0]
    g = out_ref.shape[0]
    agg = agg_ref[...]
    u = h_ref[...] + agg[:n] + agg[n:]
    a = jnp.dot(u, w1_ref[...], preferred_element_type=jnp.float32)
    a = _bn_relu(a + v1_ref[0:1], v1_ref[1:2], v1_ref[2:3])
    a = jnp.dot(a, w2_ref[...], preferred_element_type=jnp.float32)
    h3 = _bn_relu(a + v2_ref[0:1], v2_ref[1:2], v2_ref[2:3])
    gid = lax.broadcasted_iota(jnp.int32, (g, n), 0)
    p = (batch_ref[...] == gid).astype(jnp.float32)
    sums = jnp.dot(p, h3, preferred_element_type=jnp.float32)
    counts = jnp.sum(p, axis=1, keepdims=True)
    mean = sums / jnp.maximum(counts, 1.0)
    out_ref[...] = jnp.dot(mean, wc_ref[...],
                           preferred_element_type=jnp.float32) + bc_ref[...]


def _conv_args(p):
    w1 = p["W1"]
    v1 = jnp.stack([p["b1"], p["g1"], p["be1"]])
    w2 = p["W2"]
    v2 = jnp.stack([p["b2"], p["g2"], p["be2"]])
    return w1, v1, w2, v2


def kernel(x, edge_index, batch, params):
    n, d = x.shape
    e = edge_index.shape[1]
    g = 64
    c_out = params["clf_b"].shape[0]

    src = edge_index[0]
    dst = edge_index[1]
    seg_sum = _make_seg_sum(n, d, e)

    h = x
    convs = [params["conv1"], params["conv2"], params["conv3"]]
    for li in (0, 1):
        aggp = seg_sum(h, src, dst)
        w1, v1, w2, v2 = _conv_args(convs[li])
        h = pl.pallas_call(
            _mlp_body,
            out_shape=jax.ShapeDtypeStruct((n, w2.shape[1]), jnp.float32),
        )(h, aggp, w1, v1, w2, v2)

    aggp = seg_sum(h, src, dst)
    w1, v1, w2, v2 = _conv_args(convs[2])
    out = pl.pallas_call(
        _mlp_pool_body,
        out_shape=jax.ShapeDtypeStruct((g, c_out), jnp.float32),
    )(h, aggp, w1, v1, w2, v2, batch.reshape(1, n),
      params["clf_W"], params["clf_b"].reshape(1, c_out))
    return out


# speedup vs baseline: 8.4999x; 1.2482x over previous
import functools

import jax
import jax.numpy as jnp
from jax import lax
from jax.experimental import pallas as pl
from jax.experimental.pallas import tpu as pltpu
from jax.experimental.pallas import tpu_sc as plsc

_NC = 2
_NS = 16
_W = 80


def _make_seg_sum(n, d, e):
    nworker = _NC * _NS
    ept = e // nworker
    wpt = ept // _W
    zc = 80
    nchunks = n // zc
    nz = -(-nchunks // _NS)
    assert ept % _W == 0 and n % zc == 0 and wpt % 2 == 1

    mesh = plsc.VectorSubcoreMesh(
        core_axis_name="c", subcore_axis_name="s",
        num_cores=_NC, num_subcores=_NS)

    @functools.partial(
        pl.kernel,
        out_type=jax.ShapeDtypeStruct((_NC * n, d), jnp.float32),
        mesh=mesh,
        scratch_types=[
            pltpu.VMEM((ept,), jnp.int32),
            pltpu.VMEM((_W,), jnp.int32),
            pltpu.VMEM((_W,), jnp.int32),
            pltpu.VMEM((_W, d), jnp.float32),
            pltpu.VMEM((_W, d), jnp.float32),
            pltpu.VMEM((zc, d), jnp.float32),
            pltpu.VMEM_SHARED((n, d), jnp.float32),
            pltpu.SemaphoreType.DMA,
            pltpu.SemaphoreType.DMA,
            pltpu.SemaphoreType.DMA,
            pltpu.SemaphoreType.DMA,
        ],
    )
    def seg_sum(x_hbm, src_hbm, dst_hbm, out_hbm,
                sidx, dwin0, dwin1, rows0, rows1, zbuf, acc,
                sem0, sem1, dsem0, dsem1):
        c = lax.axis_index("c")
        s = lax.axis_index("s")
        wid = c * _NS + s
        ebase = wid * ept

        pltpu.sync_copy(src_hbm.at[pl.ds(ebase, ept)], sidx)

        z16 = jnp.zeros((16,), jnp.float32)

        def zrow(i, carry):
            for j in range(d // 16):
                zbuf[i, pl.ds(j * 16, 16)] = z16
            return carry

        lax.fori_loop(0, zc, zrow, 0)
        for k in range(nz):
            cid = s + k * _NS

            @pl.when(cid < nchunks)
            def _():
                pltpu.sync_copy(zbuf, acc.at[pl.ds(cid * zc, zc)])
        plsc.subcore_barrier()

        def gather(w, rows, dwin, sem, dsem):
            pltpu.async_copy(
                x_hbm.at[sidx.at[pl.ds(w * _W, _W)]], rows, sem)
            pltpu.async_copy(
                dst_hbm.at[pl.ds(ebase + w * _W, _W)], dwin, dsem)

        def wait_gather(w, rows, dwin, sem, dsem):
            pltpu.make_async_copy(
                x_hbm.at[sidx.at[pl.ds(w * _W, _W)]], rows, sem).wait()
            pltpu.make_async_copy(
                dst_hbm.at[pl.ds(ebase + w * _W, _W)], dwin, dsem).wait()

        def scatter_add(rows, dwin):
            pltpu.sync_copy(rows, acc.at[dwin], add=True)

        def body(i, carry):
            w0 = 2 * i
            gather(w0, rows0, dwin0, sem0, dsem0)
            gather(w0 + 1, rows1, dwin1, sem1, dsem1)
            wait_gather(w0, rows0, dwin0, sem0, dsem0)
            scatter_add(rows0, dwin0)
            wait_gather(w0 + 1, rows1, dwin1, sem1, dsem1)
            scatter_add(rows1, dwin1)
            return carry

        lax.fori_loop(0, wpt // 2, body, 0)
        if wpt % 2:
            gather(wpt - 1, rows0, dwin0, sem0, dsem0)
            wait_gather(wpt - 1, rows0, dwin0, sem0, dsem0)
            scatter_add(rows0, dwin0)

        plsc.subcore_barrier()
        for k in range(nz):
            cid = s + k * _NS

            @pl.when(cid < nchunks)
            def _():
                pltpu.sync_copy(acc.at[pl.ds(cid * zc, zc)],
                                out_hbm.at[pl.ds(c * n + cid * zc, zc)])

    return seg_sum


def _bn_relu(a, g, b):
    m = jnp.mean(a, axis=0, keepdims=True)
    v = jnp.mean((a - m) ** 2, axis=0, keepdims=True)
    return jnp.maximum(g * (a - m) * lax.rsqrt(v + 1e-5) + b, 0.0)


def _mlp_body(h_ref, agg_ref, w1_ref, v1_ref, w2_ref, v2_ref, out_ref):
    n = h_ref.shape[0]
    agg = agg_ref[...]
    u = h_ref[...] + agg[:n] + agg[n:]
    a = jnp.dot(u, w1_ref[...], preferred_element_type=jnp.float32)
    a = _bn_relu(a + v1_ref[0:1], v1_ref[1:2], v1_ref[2:3])
    a = jnp.dot(a, w2_ref[...], preferred_element_type=jnp.float32)
    out_ref[...] = _bn_relu(a + v2_ref[0:1], v2_ref[1:2], v2_ref[2:3])


def _mlp_pool_body(h_ref, agg_ref, w1_ref, v1_ref, w2_ref, v2_ref,
                   batch_ref, wc_ref, bc_ref, out_ref):
    n = h_ref.shape[0]
    g = out_ref.shape[0]
    agg = agg_ref[...]
    u = h_ref[...] + agg[:n] + agg[n:]
    a = jnp.dot(u, w1_ref[...], preferred_element_type=jnp.float32)
    a = _bn_relu(a + v1_ref[0:1], v1_ref[1:2], v1_ref[2:3])
    a = jnp.dot(a, w2_ref[...], preferred_element_type=jnp.float32)
    h3 = _bn_relu(a + v2_ref[0:1], v2_ref[1:2], v2_ref[2:3])
    gid = lax.broadcasted_iota(jnp.int32, (g, n), 0)
    p = (batch_ref[...] == gid).astype(jnp.float32)
    sums = jnp.dot(p, h3, preferred_element_type=jnp.float32)
    counts = jnp.sum(p, axis=1, keepdims=True)
    mean = sums / jnp.maximum(counts, 1.0)
    out_ref[...] = jnp.dot(mean, wc_ref[...],
                           preferred_element_type=jnp.float32) + bc_ref[...]


def _conv_args(p):
    w1 = p["W1"]
    v1 = jnp.stack([p["b1"], p["g1"], p["be1"]])
    w2 = p["W2"]
    v2 = jnp.stack([p["b2"], p["g2"], p["be2"]])
    return w1, v1, w2, v2


def kernel(x, edge_index, batch, params):
    n, d = x.shape
    e = edge_index.shape[1]
    g = 64
    c_out = params["clf_b"].shape[0]

    src = edge_index[0]
    dst = edge_index[1]
    seg_sum = _make_seg_sum(n, d, e)

    h = x
    convs = [params["conv1"], params["conv2"], params["conv3"]]
    for li in (0, 1):
        aggp = seg_sum(h, src, dst)
        w1, v1, w2, v2 = _conv_args(convs[li])
        h = pl.pallas_call(
            _mlp_body,
            out_shape=jax.ShapeDtypeStruct((n, w2.shape[1]), jnp.float32),
        )(h, aggp, w1, v1, w2, v2)

    aggp = seg_sum(h, src, dst)
    w1, v1, w2, v2 = _conv_args(convs[2])
    out = pl.pallas_call(
        _mlp_pool_body,
        out_shape=jax.ShapeDtypeStruct((g, c_out), jnp.float32),
    )(h, aggp, w1, v1, w2, v2, batch.reshape(1, n),
      params["clf_W"], params["clf_b"].reshape(1, c_out))
    return out
